# baseline (speedup 1.0000x reference)
import functools

import jax
import jax.numpy as jnp
from jax import lax
from jax.experimental import pallas as pl
from jax.experimental.pallas import tpu as pltpu
from jax.experimental.pallas import tpu_sc as plsc

EPS = 1e-06
M_TOTAL = 8 * 512 * 512
K_KEEP = int(M_TOTAL * 0.75)
KF = float(K_KEEP)
ROWS = 2048
COLS = 1024
GRID = 8
BLK = ROWS // GRID

H = 4224
CH_H = H // 16
NW = 32
WPW = M_TOTAL // NW
CH_DATA = WPW // 16

_mesh = plsc.VectorSubcoreMesh(core_axis_name="c", subcore_axis_name="s")
_cp = pltpu.CompilerParams(needs_layout_passes=False)


def _lanes():
    return lax.iota(jnp.int32, 16)


def _scalar(x):
    return jnp.max(x) if getattr(x, "ndim", 0) else x


def _tc_body(x_ref, t_ref, bits_ref, scal_ref, acc_ref):
    i = pl.program_id(0)
    x = x_ref[...]
    t = t_ref[...]
    posf = (t == 1).astype(jnp.float32)
    negf = 1.0 - posf

    e = jnp.exp(-jnp.abs(x))
    log1pe = jnp.log1p(e)
    ls_pos = jnp.minimum(x, 0.0) - log1pe
    ls_neg = jnp.minimum(-x, 0.0) - log1pe
    sig = jnp.where(x >= 0.0, 1.0 / (1.0 + e), e / (1.0 + e))
    pt = jnp.clip(sig, EPS, 1.0 - EPS)

    pos_loss = -jnp.sqrt(jnp.sqrt(1.0 - pt)) * ls_pos * posf
    neg_loss = jnp.abs(-(pt * pt) * ls_neg * negf)
    bits_ref[...] = lax.bitcast_convert_type(neg_loss, jnp.int32)

    p_sum = jnp.sum(pos_loss)
    p_cnt = jnp.sum(posf)
    n_cnt = jnp.sum(negf)

    @pl.when(i == 0)
    def _():
        acc_ref[0] = p_sum
        acc_ref[1] = p_cnt
        acc_ref[2] = n_cnt

    @pl.when(i > 0)
    def _():
        acc_ref[0] += p_sum
        acc_ref[1] += p_cnt
        acc_ref[2] += n_cnt

    @pl.when(i == GRID - 1)
    def _():
        scal_ref[0, 0] = acc_ref[0]
        scal_ref[0, 1] = acc_ref[1]
        scal_ref[0, 2] = acc_ref[2]
        for j in range(3, 16):
            scal_ref[0, j] = 0.0


def _tc_stage(x, t):
    return pl.pallas_call(
        _tc_body,
        grid=(GRID,),
        in_specs=[
            pl.BlockSpec((BLK, COLS), lambda i: (i, 0)),
            pl.BlockSpec((BLK, COLS), lambda i: (i, 0)),
        ],
        out_shape=[
            jax.ShapeDtypeStruct((ROWS, COLS), jnp.int32),
            jax.ShapeDtypeStruct((1, 16), jnp.float32),
        ],
        out_specs=[
            pl.BlockSpec((BLK, COLS), lambda i: (i, 0)),
            pl.BlockSpec(memory_space=pltpu.SMEM),
        ],
        scratch_shapes=[pltpu.SMEM((3,), jnp.float32)],
    )(x, t)


def _zero_hists(cnt_v, sum_v, iota_v):
    @pl.loop(0, CH_H)
    def _(i):
        z = jnp.zeros((16,), jnp.float32)
        cnt_v[pl.ds(i * 16, 16)] = z
        sum_v[pl.ds(i * 16, 16)] = z
        iota_v[pl.ds(i * 16, 16)] = _lanes() + i * 16


def _hist_scatter(data_v, cnt_v, sum_v, idx_fn):
    ones = jnp.ones((16,), jnp.float32)

    @pl.loop(0, CH_DATA, unroll=8)
    def _(i):
        v = data_v[pl.ds(i * 16, 16)]
        idx = idx_fn(v)
        f = plsc.bitcast(v, jnp.float32)
        plsc.addupdate_scatter(cnt_v, [idx], ones)
        plsc.addupdate_scatter(sum_v, [idx], f)


def _merge_and_emit(c, s, cnt_v, sum_v, iota_v, sh_cnt, sh_sum,
                    out_cnt, out_sum):
    pltpu.sync_copy(cnt_v, sh_cnt.at[iota_v], add=True)
    pltpu.sync_copy(sum_v, sh_sum.at[iota_v], add=True)
    plsc.subcore_barrier()

    @pl.when(s == 0)
    def _():
        pltpu.sync_copy(sh_cnt, out_cnt.at[c])
        pltpu.sync_copy(sh_sum, out_sum.at[c])


@functools.partial(
    pl.kernel, mesh=_mesh, compiler_params=_cp,
    out_type=[jax.ShapeDtypeStruct((2, H), jnp.float32),
              jax.ShapeDtypeStruct((2, H), jnp.float32)],
    scratch_types=[
        pltpu.VMEM((WPW,), jnp.int32),
        pltpu.VMEM((H,), jnp.float32),
        pltpu.VMEM((H,), jnp.float32),
        pltpu.VMEM((H,), jnp.int32),
        pltpu.VMEM_SHARED((H,), jnp.float32),
        pltpu.VMEM_SHARED((H,), jnp.float32),
    ])
def _sc_hist1(bits_hbm, out_cnt, out_sum,
              data_v, cnt_v, sum_v, iota_v, sh_cnt, sh_sum):
    c = lax.axis_index("c")
    s = lax.axis_index("s")
    w = c * 16 + s
    pltpu.sync_copy(bits_hbm.at[pl.ds(w * WPW, WPW)], data_v)
    _zero_hists(cnt_v, sum_v, iota_v)

    @pl.when(s == 0)
    def _():
        pltpu.sync_copy(cnt_v, sh_cnt)
        pltpu.sync_copy(sum_v, sh_sum)

    plsc.subcore_barrier()
    _hist_scatter(data_v, cnt_v, sum_v,
                  lambda v: jnp.right_shift(v, 19))
    _merge_and_emit(c, s, cnt_v, sum_v, iota_v, sh_cnt, sh_sum,
                    out_cnt, out_sum)


def _scan_for_bin(buf_cnt2d, threshold, nchunks):
    def step(i, carry):
        run, b, cb = carry
        ch = buf_cnt2d[0, pl.ds(i * 16, 16)] + buf_cnt2d[1, pl.ds(i * 16, 16)]
        cs = plsc.cumsum(ch)
        tot = run + jnp.max(cs)
        hitmask = (cs + run) >= threshold
        hit = (run < threshold) & (tot >= threshold)
        lane = _scalar(plsc.all_reduce_ffs(hitmask))
        excl_c = jnp.sum(jnp.where(_lanes() == lane, cs - ch, 0.0))
        b_n = jnp.where(hit, i * 16 + lane, b)
        cb_n = jnp.where(hit, run + excl_c, cb)
        return tot, b_n, cb_n

    return lax.fori_loop(0, nchunks, step,
                         (jnp.float32(0.0), jnp.int32(0), jnp.float32(0.0)))


@functools.partial(
    pl.kernel, mesh=_mesh, compiler_params=_cp,
    out_type=[jax.ShapeDtypeStruct((2, H), jnp.float32),
              jax.ShapeDtypeStruct((2, H), jnp.float32)],
    scratch_types=[
        pltpu.VMEM((WPW,), jnp.int32),
        pltpu.VMEM((2, H), jnp.float32),
        pltpu.VMEM((H,), jnp.float32),
        pltpu.VMEM((H,), jnp.float32),
        pltpu.VMEM((H,), jnp.int32),
        pltpu.VMEM_SHARED((H,), jnp.float32),
        pltpu.VMEM_SHARED((H,), jnp.float32),
    ])
def _sc_hist2(bits_hbm, h1cnt_hbm, out_cnt, out_sum,
              data_v, h1_v, cnt_v, sum_v, iota_v, sh_cnt, sh_sum):
    c = lax.axis_index("c")
    s = lax.axis_index("s")
    w = c * 16 + s
    pltpu.sync_copy(bits_hbm.at[pl.ds(w * WPW, WPW)], data_v)
    pltpu.sync_copy(h1cnt_hbm, h1_v)
    _zero_hists(cnt_v, sum_v, iota_v)

    @pl.when(s == 0)
    def _():
        pltpu.sync_copy(cnt_v, sh_cnt)
        pltpu.sync_copy(sum_v, sh_sum)

    _, b, _ = _scan_for_bin(h1_v, KF, 256)
    plsc.subcore_barrier()
    _hist_scatter(
        data_v, cnt_v, sum_v,
        lambda v: jnp.where(jnp.right_shift(v, 19) == b,
                            jnp.bitwise_and(jnp.right_shift(v, 7), 4095),
                            4096))
    _merge_and_emit(c, s, cnt_v, sum_v, iota_v, sh_cnt, sh_sum,
                    out_cnt, out_sum)


@functools.partial(
    pl.kernel, mesh=_mesh, compiler_params=_cp,
    out_type=jax.ShapeDtypeStruct((16,), jnp.float32),
    scratch_types=[
        pltpu.VMEM((2, H), jnp.float32),
        pltpu.VMEM((2, H), jnp.float32),
        pltpu.VMEM((2, H), jnp.float32),
        pltpu.VMEM((2, H), jnp.float32),
        pltpu.VMEM((16,), jnp.float32),
        pltpu.VMEM((16,), jnp.float32),
    ])
def _sc_finalize(h1c_hbm, h1s_hbm, h2c_hbm, h2s_hbm, scal_hbm, out_hbm,
                 c1_v, s1_v, c2_v, s2_v, scal_v, out_v):
    c = lax.axis_index("c")
    s = lax.axis_index("s")

    @pl.when((c == 0) & (s == 0))
    def _():
        pltpu.sync_copy(h1c_hbm, c1_v)
        pltpu.sync_copy(h1s_hbm, s1_v)
        pltpu.sync_copy(h2c_hbm, c2_v)
        pltpu.sync_copy(h2s_hbm, s2_v)
        pltpu.sync_copy(scal_hbm, scal_v)

        def scan_full(cbuf, sbuf, threshold):
            def step(i, carry):
                run_c, run_s, b, cb, sb, cbin, sbin = carry
                chc = cbuf[0, pl.ds(i * 16, 16)] + cbuf[1, pl.ds(i * 16, 16)]
                chs = sbuf[0, pl.ds(i * 16, 16)] + sbuf[1, pl.ds(i * 16, 16)]
                csc = plsc.cumsum(chc)
                css = plsc.cumsum(chs)
                tot_c = run_c + jnp.max(csc)
                tot_s = run_s + jnp.max(css)
                hitmask = (csc + run_c) >= threshold
                hit = (run_c < threshold) & (tot_c >= threshold)
                lane = _scalar(plsc.all_reduce_ffs(hitmask))
                sel = _lanes() == lane
                excl_c = jnp.sum(jnp.where(sel, csc - chc, 0.0))
                excl_s = jnp.sum(jnp.where(sel, css - chs, 0.0))
                bin_c = jnp.sum(jnp.where(sel, chc, 0.0))
                bin_s = jnp.sum(jnp.where(sel, chs, 0.0))
                b_n = jnp.where(hit, i * 16 + lane, b)
                cb_n = jnp.where(hit, run_c + excl_c, cb)
                sb_n = jnp.where(hit, run_s + excl_s, sb)
                cbin_n = jnp.where(hit, bin_c, cbin)
                sbin_n = jnp.where(hit, bin_s, sbin)
                return tot_c, tot_s, b_n, cb_n, sb_n, cbin_n, sbin_n

            z = jnp.float32(0.0)
            return lax.fori_loop(0, 256, step,
                                 (z, z, jnp.int32(0), z, z, z, z))

        _, _, _, cb1, sb1, _, _ = scan_full(c1_v, s1_v, KF)
        k2p = KF - cb1
        _, _, _, cb2, sb2, cbin, sbin = scan_full(c2_v, s2_v, k2p)

        z16 = jnp.zeros((16,), jnp.float32)
        cbin16 = z16 + cbin
        sbin16 = z16 + sbin
        mean16 = jnp.where(cbin16 > 0.0,
                           sbin16 / jnp.maximum(cbin16, 1.0), 0.0)
        bottom16 = (z16 + sb1) + (z16 + sb2) + (z16 + (k2p - cb2)) * mean16

        sv = scal_v[...]
        pos_sum = jnp.sum(jnp.where(_lanes() == 0, sv, 0.0))
        pos_cnt = jnp.sum(jnp.where(_lanes() == 1, sv, 0.0))
        neg_cnt = jnp.sum(jnp.where(_lanes() == 2, sv, 0.0))
        loss16 = ((z16 + pos_sum) / (z16 + pos_cnt + EPS)
                  + bottom16 / (z16 + neg_cnt + EPS))
        out_v[...] = loss16
        pltpu.sync_copy(out_v, out_hbm)


def kernel(output, target):
    x = output.astype(jnp.float32).reshape(ROWS, COLS)
    t = target.reshape(ROWS, COLS)
    bits2d, scal = _tc_stage(x, t)
    bits = bits2d.reshape(M_TOTAL)
    h1c, h1s = _sc_hist1(bits)
    h2c, h2s = _sc_hist2(bits, h1c)
    out = _sc_finalize(h1c, h1s, h2c, h2s, scal.reshape(16))
    return out[0]


# speedup vs baseline: 1.1069x; 1.1069x over previous
import functools

import jax
import jax.numpy as jnp
from jax import lax
from jax.experimental import pallas as pl
from jax.experimental.pallas import tpu as pltpu
from jax.experimental.pallas import tpu_sc as plsc

EPS = 1e-06
M_TOTAL = 8 * 512 * 512
K_KEEP = int(M_TOTAL * 0.75)
KF = float(K_KEEP)
ROWS = 2048
COLS = 1024
GRID = 8
BLK = ROWS // GRID

H = 4224
CH_H = H // 16
NW = 32
WPW = M_TOTAL // NW
CH_DATA = WPW // 16

_mesh = plsc.VectorSubcoreMesh(core_axis_name="c", subcore_axis_name="s")
_cp = pltpu.CompilerParams(needs_layout_passes=False)


def _lanes():
    return lax.iota(jnp.int32, 16)


def _scalar(x):
    return jnp.max(x) if getattr(x, "ndim", 0) else x


def _tc_body(x_ref, t_ref, bits_ref, scal_ref, acc_ref):
    i = pl.program_id(0)
    x = x_ref[...]
    t = t_ref[...]
    posf = (t == 1).astype(jnp.float32)
    negf = 1.0 - posf

    e = jnp.exp(-jnp.abs(x))
    log1pe = jnp.log1p(e)
    ls_pos = jnp.minimum(x, 0.0) - log1pe
    ls_neg = jnp.minimum(-x, 0.0) - log1pe
    sig = jnp.where(x >= 0.0, 1.0 / (1.0 + e), e / (1.0 + e))
    pt = jnp.clip(sig, EPS, 1.0 - EPS)

    pos_loss = -jnp.sqrt(jnp.sqrt(1.0 - pt)) * ls_pos * posf
    neg_loss = jnp.abs(-(pt * pt) * ls_neg * negf)
    bits_ref[...] = lax.bitcast_convert_type(neg_loss, jnp.int32)

    p_sum = jnp.sum(pos_loss)
    p_cnt = jnp.sum(posf)
    n_cnt = jnp.sum(negf)

    @pl.when(i == 0)
    def _():
        acc_ref[0] = p_sum
        acc_ref[1] = p_cnt
        acc_ref[2] = n_cnt

    @pl.when(i > 0)
    def _():
        acc_ref[0] += p_sum
        acc_ref[1] += p_cnt
        acc_ref[2] += n_cnt

    @pl.when(i == GRID - 1)
    def _():
        scal_ref[0, 0] = acc_ref[0]
        scal_ref[0, 1] = acc_ref[1]
        scal_ref[0, 2] = acc_ref[2]
        for j in range(3, 16):
            scal_ref[0, j] = 0.0


def _tc_stage(x, t):
    return pl.pallas_call(
        _tc_body,
        grid=(GRID,),
        in_specs=[
            pl.BlockSpec((BLK, COLS), lambda i: (i, 0)),
            pl.BlockSpec((BLK, COLS), lambda i: (i, 0)),
        ],
        out_shape=[
            jax.ShapeDtypeStruct((ROWS, COLS), jnp.int32),
            jax.ShapeDtypeStruct((1, 16), jnp.float32),
        ],
        out_specs=[
            pl.BlockSpec((BLK, COLS), lambda i: (i, 0)),
            pl.BlockSpec(memory_space=pltpu.SMEM),
        ],
        scratch_shapes=[pltpu.SMEM((3,), jnp.float32)],
    )(x, t)


def _zero_hists(cnt_v, sum_v, iota_v):
    @pl.loop(0, CH_H)
    def _(i):
        z = jnp.zeros((16,), jnp.float32)
        cnt_v[pl.ds(i * 16, 16)] = z
        sum_v[pl.ds(i * 16, 16)] = z
        iota_v[pl.ds(i * 16, 16)] = _lanes() + i * 16


def _hist_scatter(data_v, cnt_v, sum_v, idx_fn):
    ones = jnp.ones((16,), jnp.float32)

    @plsc.parallel_loop(0, CH_DATA, unroll=8)
    def _(i):
        v = data_v[pl.ds(i * 16, 16)]
        idx = idx_fn(v)
        f = plsc.bitcast(v, jnp.float32)
        plsc.addupdate_scatter(cnt_v, [idx], ones)
        plsc.addupdate_scatter(sum_v, [idx], f)


def _merge_and_emit(c, s, cnt_v, sum_v, iota_v, sh_cnt, sh_sum,
                    out_cnt, out_sum):
    pltpu.sync_copy(cnt_v, sh_cnt.at[iota_v], add=True)
    pltpu.sync_copy(sum_v, sh_sum.at[iota_v], add=True)
    plsc.subcore_barrier()

    @pl.when(s == 0)
    def _():
        pltpu.sync_copy(sh_cnt, out_cnt.at[c])
        pltpu.sync_copy(sh_sum, out_sum.at[c])


@functools.partial(
    pl.kernel, mesh=_mesh, compiler_params=_cp,
    out_type=[jax.ShapeDtypeStruct((2, H), jnp.float32),
              jax.ShapeDtypeStruct((2, H), jnp.float32)],
    scratch_types=[
        pltpu.VMEM((WPW,), jnp.int32),
        pltpu.VMEM((H,), jnp.float32),
        pltpu.VMEM((H,), jnp.float32),
        pltpu.VMEM((H,), jnp.int32),
        pltpu.VMEM_SHARED((H,), jnp.float32),
        pltpu.VMEM_SHARED((H,), jnp.float32),
    ])
def _sc_hist1(bits_hbm, out_cnt, out_sum,
              data_v, cnt_v, sum_v, iota_v, sh_cnt, sh_sum):
    c = lax.axis_index("c")
    s = lax.axis_index("s")
    w = c * 16 + s
    pltpu.sync_copy(bits_hbm.at[pl.ds(w * WPW, WPW)], data_v)
    _zero_hists(cnt_v, sum_v, iota_v)

    @pl.when(s == 0)
    def _():
        pltpu.sync_copy(cnt_v, sh_cnt)
        pltpu.sync_copy(sum_v, sh_sum)

    plsc.subcore_barrier()
    _hist_scatter(data_v, cnt_v, sum_v,
                  lambda v: jnp.right_shift(v, 19))
    _merge_and_emit(c, s, cnt_v, sum_v, iota_v, sh_cnt, sh_sum,
                    out_cnt, out_sum)


def _scan_for_bin(buf_cnt2d, threshold, nchunks):
    def step(i, carry):
        run, b, cb = carry
        ch = buf_cnt2d[0, pl.ds(i * 16, 16)] + buf_cnt2d[1, pl.ds(i * 16, 16)]
        cs = plsc.cumsum(ch)
        tot = run + jnp.max(cs)
        hitmask = (cs + run) >= threshold
        hit = (run < threshold) & (tot >= threshold)
        lane = _scalar(plsc.all_reduce_ffs(hitmask))
        excl_c = jnp.sum(jnp.where(_lanes() == lane, cs - ch, 0.0))
        b_n = jnp.where(hit, i * 16 + lane, b)
        cb_n = jnp.where(hit, run + excl_c, cb)
        return tot, b_n, cb_n

    return lax.fori_loop(0, nchunks, step,
                         (jnp.float32(0.0), jnp.int32(0), jnp.float32(0.0)))


@functools.partial(
    pl.kernel, mesh=_mesh, compiler_params=_cp,
    out_type=[jax.ShapeDtypeStruct((2, H), jnp.float32),
              jax.ShapeDtypeStruct((2, H), jnp.float32)],
    scratch_types=[
        pltpu.VMEM((WPW,), jnp.int32),
        pltpu.VMEM((2, H), jnp.float32),
        pltpu.VMEM((H,), jnp.float32),
        pltpu.VMEM((H,), jnp.float32),
        pltpu.VMEM((H,), jnp.int32),
        pltpu.VMEM_SHARED((H,), jnp.float32),
        pltpu.VMEM_SHARED((H,), jnp.float32),
    ])
def _sc_hist2(bits_hbm, h1cnt_hbm, out_cnt, out_sum,
              data_v, h1_v, cnt_v, sum_v, iota_v, sh_cnt, sh_sum):
    c = lax.axis_index("c")
    s = lax.axis_index("s")
    w = c * 16 + s
    pltpu.sync_copy(bits_hbm.at[pl.ds(w * WPW, WPW)], data_v)
    pltpu.sync_copy(h1cnt_hbm, h1_v)
    _zero_hists(cnt_v, sum_v, iota_v)

    @pl.when(s == 0)
    def _():
        pltpu.sync_copy(cnt_v, sh_cnt)
        pltpu.sync_copy(sum_v, sh_sum)

    _, b, _ = _scan_for_bin(h1_v, KF, 256)
    plsc.subcore_barrier()
    _hist_scatter(
        data_v, cnt_v, sum_v,
        lambda v: jnp.where(jnp.right_shift(v, 19) == b,
                            jnp.bitwise_and(jnp.right_shift(v, 7), 4095),
                            4096))
    _merge_and_emit(c, s, cnt_v, sum_v, iota_v, sh_cnt, sh_sum,
                    out_cnt, out_sum)


@functools.partial(
    pl.kernel, mesh=_mesh, compiler_params=_cp,
    out_type=jax.ShapeDtypeStruct((16,), jnp.float32),
    scratch_types=[
        pltpu.VMEM((2, H), jnp.float32),
        pltpu.VMEM((2, H), jnp.float32),
        pltpu.VMEM((2, H), jnp.float32),
        pltpu.VMEM((2, H), jnp.float32),
        pltpu.VMEM((16,), jnp.float32),
        pltpu.VMEM((16,), jnp.float32),
    ])
def _sc_finalize(h1c_hbm, h1s_hbm, h2c_hbm, h2s_hbm, scal_hbm, out_hbm,
                 c1_v, s1_v, c2_v, s2_v, scal_v, out_v):
    c = lax.axis_index("c")
    s = lax.axis_index("s")

    @pl.when((c == 0) & (s == 0))
    def _():
        pltpu.sync_copy(h1c_hbm, c1_v)
        pltpu.sync_copy(h1s_hbm, s1_v)
        pltpu.sync_copy(h2c_hbm, c2_v)
        pltpu.sync_copy(h2s_hbm, s2_v)
        pltpu.sync_copy(scal_hbm, scal_v)

        def scan_full(cbuf, sbuf, threshold):
            def step(i, carry):
                run_c, run_s, b, cb, sb, cbin, sbin = carry
                chc = cbuf[0, pl.ds(i * 16, 16)] + cbuf[1, pl.ds(i * 16, 16)]
                chs = sbuf[0, pl.ds(i * 16, 16)] + sbuf[1, pl.ds(i * 16, 16)]
                csc = plsc.cumsum(chc)
                css = plsc.cumsum(chs)
                tot_c = run_c + jnp.max(csc)
                tot_s = run_s + jnp.max(css)
                hitmask = (csc + run_c) >= threshold
                hit = (run_c < threshold) & (tot_c >= threshold)
                lane = _scalar(plsc.all_reduce_ffs(hitmask))
                sel = _lanes() == lane
                excl_c = jnp.sum(jnp.where(sel, csc - chc, 0.0))
                excl_s = jnp.sum(jnp.where(sel, css - chs, 0.0))
                bin_c = jnp.sum(jnp.where(sel, chc, 0.0))
                bin_s = jnp.sum(jnp.where(sel, chs, 0.0))
                b_n = jnp.where(hit, i * 16 + lane, b)
                cb_n = jnp.where(hit, run_c + excl_c, cb)
                sb_n = jnp.where(hit, run_s + excl_s, sb)
                cbin_n = jnp.where(hit, bin_c, cbin)
                sbin_n = jnp.where(hit, bin_s, sbin)
                return tot_c, tot_s, b_n, cb_n, sb_n, cbin_n, sbin_n

            z = jnp.float32(0.0)
            return lax.fori_loop(0, 256, step,
                                 (z, z, jnp.int32(0), z, z, z, z))

        _, _, _, cb1, sb1, _, _ = scan_full(c1_v, s1_v, KF)
        k2p = KF - cb1
        _, _, _, cb2, sb2, cbin, sbin = scan_full(c2_v, s2_v, k2p)

        z16 = jnp.zeros((16,), jnp.float32)
        cbin16 = z16 + cbin
        sbin16 = z16 + sbin
        mean16 = jnp.where(cbin16 > 0.0,
                           sbin16 / jnp.maximum(cbin16, 1.0), 0.0)
        bottom16 = (z16 + sb1) + (z16 + sb2) + (z16 + (k2p - cb2)) * mean16

        sv = scal_v[...]
        pos_sum = jnp.sum(jnp.where(_lanes() == 0, sv, 0.0))
        pos_cnt = jnp.sum(jnp.where(_lanes() == 1, sv, 0.0))
        neg_cnt = jnp.sum(jnp.where(_lanes() == 2, sv, 0.0))
        loss16 = ((z16 + pos_sum) / (z16 + pos_cnt + EPS)
                  + bottom16 / (z16 + neg_cnt + EPS))
        out_v[...] = loss16
        pltpu.sync_copy(out_v, out_hbm)


def kernel(output, target):
    x = output.astype(jnp.float32).reshape(ROWS, COLS)
    t = target.reshape(ROWS, COLS)
    bits2d, scal = _tc_stage(x, t)
    bits = bits2d.reshape(M_TOTAL)
    h1c, h1s = _sc_hist1(bits)
    h2c, h2s = _sc_hist2(bits, h1c)
    out = _sc_finalize(h1c, h1s, h2c, h2s, scal.reshape(16))
    return out[0]


# speedup vs baseline: 2.8592x; 2.5830x over previous
import functools

import jax
import jax.numpy as jnp
from jax import lax
from jax.experimental import pallas as pl
from jax.experimental.pallas import tpu as pltpu
from jax.experimental.pallas import tpu_sc as plsc

EPS = 1e-06
M_TOTAL = 8 * 512 * 512
K_KEEP = int(M_TOTAL * 0.75)
KF = float(K_KEEP)
ROWS = 2048
COLS = 1024
GRID = 8
BLK = ROWS // GRID

H = 4224
CH_H = H // 16
NW = 32
WPW = M_TOTAL // NW
CH_DATA = WPW // 16

_mesh = plsc.VectorSubcoreMesh(core_axis_name="c", subcore_axis_name="s")
_cp = pltpu.CompilerParams(needs_layout_passes=False)


def _lanes():
    return lax.iota(jnp.int32, 16)


def _scalar(x):
    return jnp.max(x) if getattr(x, "ndim", 0) else x


def _tc_body(x_ref, t_ref, bits_ref, scal_ref, acc_ref):
    i = pl.program_id(0)
    x = x_ref[...]
    t = t_ref[...]
    posf = (t == 1).astype(jnp.float32)
    negf = 1.0 - posf

    e = jnp.exp(-jnp.abs(x))
    log1pe = jnp.log1p(e)
    ls_pos = jnp.minimum(x, 0.0) - log1pe
    ls_neg = jnp.minimum(-x, 0.0) - log1pe
    sig = jnp.where(x >= 0.0, 1.0 / (1.0 + e), e / (1.0 + e))
    pt = jnp.clip(sig, EPS, 1.0 - EPS)

    pos_loss = -jnp.sqrt(jnp.sqrt(1.0 - pt)) * ls_pos * posf
    neg_loss = jnp.abs(-(pt * pt) * ls_neg * negf)
    bits_ref[...] = lax.bitcast_convert_type(neg_loss, jnp.int32)

    p_sum = jnp.sum(pos_loss)
    p_cnt = jnp.sum(posf)
    n_cnt = jnp.sum(negf)

    @pl.when(i == 0)
    def _():
        acc_ref[0] = p_sum
        acc_ref[1] = p_cnt
        acc_ref[2] = n_cnt

    @pl.when(i > 0)
    def _():
        acc_ref[0] += p_sum
        acc_ref[1] += p_cnt
        acc_ref[2] += n_cnt

    @pl.when(i == GRID - 1)
    def _():
        scal_ref[0, 0] = acc_ref[0]
        scal_ref[0, 1] = acc_ref[1]
        scal_ref[0, 2] = acc_ref[2]
        for j in range(3, 16):
            scal_ref[0, j] = 0.0


def _tc_stage(x, t):
    return pl.pallas_call(
        _tc_body,
        grid=(GRID,),
        in_specs=[
            pl.BlockSpec((BLK, COLS), lambda i: (i, 0)),
            pl.BlockSpec((BLK, COLS), lambda i: (i, 0)),
        ],
        out_shape=[
            jax.ShapeDtypeStruct((ROWS, COLS), jnp.int32),
            jax.ShapeDtypeStruct((1, 16), jnp.float32),
        ],
        out_specs=[
            pl.BlockSpec((BLK, COLS), lambda i: (i, 0)),
            pl.BlockSpec(memory_space=pltpu.SMEM),
        ],
        scratch_shapes=[pltpu.SMEM((3,), jnp.float32)],
    )(x, t)


def _zero_hists(cnt_v, sum_v, iota_v):
    @pl.loop(0, CH_H)
    def _(i):
        z = jnp.zeros((16,), jnp.float32)
        cnt_v[pl.ds(i * 16, 16)] = z
        sum_v[pl.ds(i * 16, 16)] = z
        iota_v[pl.ds(i * 16, 16)] = _lanes() + i * 16


def _hist_scatter(data_v, cnt_v, sum_v, idx_fn, mask_fn):
    ones = jnp.ones((16,), jnp.float32)

    @plsc.parallel_loop(0, CH_DATA, unroll=8,
                        carry=jnp.zeros((16,), jnp.float32))
    def zc(i, acc):
        v = data_v[pl.ds(i * 16, 16)]
        m = mask_fn(v) & (v != 0)
        idx = idx_fn(v)
        f = plsc.bitcast(v, jnp.float32)
        plsc.addupdate_scatter(cnt_v, [idx], ones, mask=m)
        plsc.addupdate_scatter(sum_v, [idx], f, mask=m)
        return acc + jnp.where(v == 0, 1.0, 0.0)

    return zc


def _merge_and_emit(c, s, cnt_v, sum_v, iota_v, sh_cnt, sh_sum,
                    out_cnt, out_sum):
    pltpu.sync_copy(cnt_v, sh_cnt.at[iota_v], add=True)
    pltpu.sync_copy(sum_v, sh_sum.at[iota_v], add=True)
    plsc.subcore_barrier()

    @pl.when(s == 0)
    def _():
        pltpu.sync_copy(sh_cnt, out_cnt.at[c])
        pltpu.sync_copy(sh_sum, out_sum.at[c])


@functools.partial(
    pl.kernel, mesh=_mesh, compiler_params=_cp,
    out_type=[jax.ShapeDtypeStruct((2, H), jnp.float32),
              jax.ShapeDtypeStruct((2, H), jnp.float32)],
    scratch_types=[
        pltpu.VMEM((WPW,), jnp.int32),
        pltpu.VMEM((H,), jnp.float32),
        pltpu.VMEM((H,), jnp.float32),
        pltpu.VMEM((H,), jnp.int32),
        pltpu.VMEM_SHARED((H,), jnp.float32),
        pltpu.VMEM_SHARED((H,), jnp.float32),
    ])
def _sc_hist1(bits_hbm, out_cnt, out_sum,
              data_v, cnt_v, sum_v, iota_v, sh_cnt, sh_sum):
    c = lax.axis_index("c")
    s = lax.axis_index("s")
    w = c * 16 + s
    pltpu.sync_copy(bits_hbm.at[pl.ds(w * WPW, WPW)], data_v)
    _zero_hists(cnt_v, sum_v, iota_v)

    @pl.when(s == 0)
    def _():
        pltpu.sync_copy(cnt_v, sh_cnt)
        pltpu.sync_copy(sum_v, sh_sum)

    plsc.subcore_barrier()
    zc = _hist_scatter(data_v, cnt_v, sum_v,
                       lambda v: jnp.right_shift(v, 19),
                       lambda v: jnp.full((16,), True))
    c0 = cnt_v[pl.ds(0, 16)]
    cnt_v[pl.ds(0, 16)] = c0 + jnp.where(_lanes() == 0, jnp.sum(zc), 0.0)
    _merge_and_emit(c, s, cnt_v, sum_v, iota_v, sh_cnt, sh_sum,
                    out_cnt, out_sum)


def _scan_for_bin(buf_cnt2d, threshold, nchunks):
    def step(i, carry):
        run, b, cb = carry
        ch = buf_cnt2d[0, pl.ds(i * 16, 16)] + buf_cnt2d[1, pl.ds(i * 16, 16)]
        cs = plsc.cumsum(ch)
        tot = run + jnp.max(cs)
        hitmask = (cs + run) >= threshold
        hit = (run < threshold) & (tot >= threshold)
        lane = _scalar(plsc.all_reduce_ffs(hitmask))
        excl_c = jnp.sum(jnp.where(_lanes() == lane, cs - ch, 0.0))
        b_n = jnp.where(hit, i * 16 + lane, b)
        cb_n = jnp.where(hit, run + excl_c, cb)
        return tot, b_n, cb_n

    return lax.fori_loop(0, nchunks, step,
                         (jnp.float32(0.0), jnp.int32(0), jnp.float32(0.0)))


@functools.partial(
    pl.kernel, mesh=_mesh, compiler_params=_cp,
    out_type=[jax.ShapeDtypeStruct((2, H), jnp.float32),
              jax.ShapeDtypeStruct((2, H), jnp.float32)],
    scratch_types=[
        pltpu.VMEM((WPW,), jnp.int32),
        pltpu.VMEM((2, H), jnp.float32),
        pltpu.VMEM((H,), jnp.float32),
        pltpu.VMEM((H,), jnp.float32),
        pltpu.VMEM((H,), jnp.int32),
        pltpu.VMEM_SHARED((H,), jnp.float32),
        pltpu.VMEM_SHARED((H,), jnp.float32),
    ])
def _sc_hist2(bits_hbm, h1cnt_hbm, out_cnt, out_sum,
              data_v, h1_v, cnt_v, sum_v, iota_v, sh_cnt, sh_sum):
    c = lax.axis_index("c")
    s = lax.axis_index("s")
    w = c * 16 + s
    pltpu.sync_copy(bits_hbm.at[pl.ds(w * WPW, WPW)], data_v)
    pltpu.sync_copy(h1cnt_hbm, h1_v)
    _zero_hists(cnt_v, sum_v, iota_v)

    @pl.when(s == 0)
    def _():
        pltpu.sync_copy(cnt_v, sh_cnt)
        pltpu.sync_copy(sum_v, sh_sum)

    _, b, _ = _scan_for_bin(h1_v, KF, 256)
    plsc.subcore_barrier()
    zc = _hist_scatter(
        data_v, cnt_v, sum_v,
        lambda v: jnp.bitwise_and(jnp.right_shift(v, 7), 4095),
        lambda v: jnp.right_shift(v, 19) == b)
    @pl.when(b == 0)
    def _():
        c0 = cnt_v[pl.ds(0, 16)]
        cnt_v[pl.ds(0, 16)] = c0 + jnp.where(_lanes() == 0, jnp.sum(zc), 0.0)

    _merge_and_emit(c, s, cnt_v, sum_v, iota_v, sh_cnt, sh_sum,
                    out_cnt, out_sum)


@functools.partial(
    pl.kernel, mesh=_mesh, compiler_params=_cp,
    out_type=jax.ShapeDtypeStruct((16,), jnp.float32),
    scratch_types=[
        pltpu.VMEM((2, H), jnp.float32),
        pltpu.VMEM((2, H), jnp.float32),
        pltpu.VMEM((2, H), jnp.float32),
        pltpu.VMEM((2, H), jnp.float32),
        pltpu.VMEM((16,), jnp.float32),
        pltpu.VMEM((16,), jnp.float32),
    ])
def _sc_finalize(h1c_hbm, h1s_hbm, h2c_hbm, h2s_hbm, scal_hbm, out_hbm,
                 c1_v, s1_v, c2_v, s2_v, scal_v, out_v):
    c = lax.axis_index("c")
    s = lax.axis_index("s")

    @pl.when((c == 0) & (s == 0))
    def _():
        pltpu.sync_copy(h1c_hbm, c1_v)
        pltpu.sync_copy(h1s_hbm, s1_v)
        pltpu.sync_copy(h2c_hbm, c2_v)
        pltpu.sync_copy(h2s_hbm, s2_v)
        pltpu.sync_copy(scal_hbm, scal_v)

        def scan_full(cbuf, sbuf, threshold):
            def step(i, carry):
                run_c, run_s, b, cb, sb, cbin, sbin = carry
                chc = cbuf[0, pl.ds(i * 16, 16)] + cbuf[1, pl.ds(i * 16, 16)]
                chs = sbuf[0, pl.ds(i * 16, 16)] + sbuf[1, pl.ds(i * 16, 16)]
                csc = plsc.cumsum(chc)
                css = plsc.cumsum(chs)
                tot_c = run_c + jnp.max(csc)
                tot_s = run_s + jnp.max(css)
                hitmask = (csc + run_c) >= threshold
                hit = (run_c < threshold) & (tot_c >= threshold)
                lane = _scalar(plsc.all_reduce_ffs(hitmask))
                sel = _lanes() == lane
                excl_c = jnp.sum(jnp.where(sel, csc - chc, 0.0))
                excl_s = jnp.sum(jnp.where(sel, css - chs, 0.0))
                bin_c = jnp.sum(jnp.where(sel, chc, 0.0))
                bin_s = jnp.sum(jnp.where(sel, chs, 0.0))
                b_n = jnp.where(hit, i * 16 + lane, b)
                cb_n = jnp.where(hit, run_c + excl_c, cb)
                sb_n = jnp.where(hit, run_s + excl_s, sb)
                cbin_n = jnp.where(hit, bin_c, cbin)
                sbin_n = jnp.where(hit, bin_s, sbin)
                return tot_c, tot_s, b_n, cb_n, sb_n, cbin_n, sbin_n

            z = jnp.float32(0.0)
            return lax.fori_loop(0, 256, step,
                                 (z, z, jnp.int32(0), z, z, z, z))

        _, _, _, cb1, sb1, _, _ = scan_full(c1_v, s1_v, KF)
        k2p = KF - cb1
        _, _, _, cb2, sb2, cbin, sbin = scan_full(c2_v, s2_v, k2p)

        z16 = jnp.zeros((16,), jnp.float32)
        cbin16 = z16 + cbin
        sbin16 = z16 + sbin
        mean16 = jnp.where(cbin16 > 0.0,
                           sbin16 / jnp.maximum(cbin16, 1.0), 0.0)
        bottom16 = (z16 + sb1) + (z16 + sb2) + (z16 + (k2p - cb2)) * mean16

        sv = scal_v[...]
        pos_sum = jnp.sum(jnp.where(_lanes() == 0, sv, 0.0))
        pos_cnt = jnp.sum(jnp.where(_lanes() == 1, sv, 0.0))
        neg_cnt = jnp.sum(jnp.where(_lanes() == 2, sv, 0.0))
        loss16 = ((z16 + pos_sum) / (z16 + pos_cnt + EPS)
                  + bottom16 / (z16 + neg_cnt + EPS))
        out_v[...] = loss16
        pltpu.sync_copy(out_v, out_hbm)


def kernel(output, target):
    x = output.astype(jnp.float32).reshape(ROWS, COLS)
    t = target.reshape(ROWS, COLS)
    bits2d, scal = _tc_stage(x, t)
    bits = bits2d.reshape(M_TOTAL)
    h1c, h1s = _sc_hist1(bits)
    h2c, h2s = _sc_hist2(bits, h1c)
    out = _sc_finalize(h1c, h1s, h2c, h2s, scal.reshape(16))
    return out[0]
